# baseline (speedup 1.0000x reference)
import functools

import jax
import jax.numpy as jnp
from jax import lax
from jax.experimental import pallas as pl
from jax.experimental.pallas import tpu as pltpu
from jax.experimental.pallas import tpu_sc as plsc

VOCAB = 1000000
D = 64
B = 16384
NEG = 20
OUT_COLS = 32

NC = 2
NS = 16
NW = NC * NS
BPW = B // NW
CHUNK = 64
NCHUNK = BPW // CHUNK


def _sc_scores(cen_idx, ctx_idx, neg_idx, cen_emb, ctx_emb):
    mesh = plsc.VectorSubcoreMesh(
        core_axis_name="c", subcore_axis_name="s",
        num_cores=NC, num_subcores=NS,
    )

    @functools.partial(
        pl.kernel,
        out_type=jax.ShapeDtypeStruct((B, OUT_COLS), jnp.float32),
        mesh=mesh,
        scratch_types=[
            pltpu.VMEM((BPW,), jnp.int32),
            pltpu.VMEM((BPW,), jnp.int32),
            pltpu.VMEM((BPW * NEG,), jnp.int32),
            pltpu.VMEM((CHUNK, D), jnp.float32),
            pltpu.VMEM((CHUNK, D), jnp.float32),
            pltpu.VMEM((CHUNK * NEG, D), jnp.float32),
            pltpu.VMEM((CHUNK, OUT_COLS), jnp.float32),
            pltpu.SemaphoreType.DMA,
        ],
        compiler_params=pltpu.CompilerParams(use_tc_tiling_on_sc=False),
    )
    def body(cen_idx_h, ctx_idx_h, neg_idx_h, cen_emb_h, ctx_emb_h, out_h,
             cid_v, xid_v, nid_v, cen_v, ctx_v, neg_v, sc_v, sem):
        wid = lax.axis_index("s") * NC + lax.axis_index("c")
        base = wid * BPW
        pltpu.sync_copy(cen_idx_h.at[pl.ds(base, BPW)], cid_v)
        pltpu.sync_copy(ctx_idx_h.at[pl.ds(base, BPW)], xid_v)
        pltpu.sync_copy(neg_idx_h.at[pl.ds(base * NEG, BPW * NEG)], nid_v)

        for g in range(NCHUNK):
            c1 = pltpu.async_copy(
                cen_emb_h.at[cid_v.at[pl.ds(g * CHUNK, CHUNK)]], cen_v, sem)
            c2 = pltpu.async_copy(
                ctx_emb_h.at[xid_v.at[pl.ds(g * CHUNK, CHUNK)]], ctx_v, sem)
            c3 = pltpu.async_copy(
                ctx_emb_h.at[nid_v.at[pl.ds(g * CHUNK * NEG, CHUNK * NEG)]],
                neg_v, sem)
            c1.wait()
            c2.wait()
            c3.wait()

            lanes = lax.iota(jnp.int32, 16)

            dnums = lax.GatherDimensionNumbers(
                offset_dims=(), collapsed_slice_dims=(0,), start_index_map=(0,))

            def shuffle(v, idx):
                return lax.gather(
                    v, idx[:, None], dnums, slice_sizes=(1,),
                    mode=lax.GatherScatterMode.PROMISE_IN_BOUNDS)

            def hsum(v):
                for sh in (8, 4, 2, 1):
                    v = v + shuffle(v, lanes ^ sh)
                return v

            def elem(b, carry):
                cregs = [cen_v[b, pl.ds(16 * k, 16)] for k in range(4)]
                p = cregs[0] * ctx_v[b, pl.ds(0, 16)]
                for k in range(1, 4):
                    p = p + cregs[k] * ctx_v[b, pl.ds(16 * k, 16)]
                v_lo = jnp.where(lanes == 0, hsum(p), 0.0)
                v_hi = jnp.zeros((16,), jnp.float32)
                for j in range(NEG):
                    r = b * NEG + j
                    q = cregs[0] * neg_v[r, pl.ds(0, 16)]
                    for k in range(1, 4):
                        q = q + cregs[k] * neg_v[r, pl.ds(16 * k, 16)]
                    s = hsum(q)
                    if j + 1 < 16:
                        v_lo = jnp.where(lanes == j + 1, s, v_lo)
                    else:
                        v_hi = jnp.where(lanes == j + 1 - 16, s, v_hi)
                sc_v[b, pl.ds(0, 16)] = v_lo
                sc_v[b, pl.ds(16, 16)] = v_hi
                return carry

            lax.fori_loop(0, CHUNK, elem, 0)
            pltpu.sync_copy(sc_v, out_h.at[pl.ds(base + g * CHUNK, CHUNK)])

    return body(cen_idx, ctx_idx, neg_idx, cen_emb, ctx_emb)


def _tc_loss(scores):
    def body(s_ref, o_ref):
        x = s_ref[...]
        col = lax.broadcasted_iota(jnp.int32, x.shape, 1)
        valid = col < (NEG + 1)
        xs = jnp.where(valid, x, 0.0)
        sg = jnp.clip(jax.nn.sigmoid(xs), 1e-10, 1.0 - 1e-10)
        contrib = jnp.where(col == 0, -jnp.log(sg), -jnp.log(1.0 - sg))
        contrib = jnp.where(valid, contrib, 0.0)
        o_ref[0, 0] = jnp.sum(contrib) / B

    return pl.pallas_call(
        body,
        out_shape=jax.ShapeDtypeStruct((1, 1), jnp.float32),
        out_specs=pl.BlockSpec(memory_space=pltpu.SMEM),
    )(scores)


@jax.jit
def kernel(center_words, context_words, negative_words, center_emb, context_emb):
    cen_idx = center_words.astype(jnp.int32)
    ctx_idx = context_words.astype(jnp.int32)
    neg_idx = negative_words.astype(jnp.int32).reshape(B * NEG)
    scores = _sc_scores(cen_idx, ctx_idx, neg_idx, center_emb, context_emb)
    loss = _tc_loss(scores)
    return loss[0, 0]
